# initial kernel scaffold (unmeasured)
import jax
import jax.numpy as jnp
from jax import lax
from jax.experimental import pallas as pl
from jax.experimental.pallas import tpu as pltpu

P = 16
COMM_DTYPE = jnp.float32


def kernel(x, w_mat):
    m_full, k_per = x.shape
    _, n = w_mat.shape
    m_per = m_full // P

    def body(x_ref, w_ref, out_ref, rbuf, sbuf, abuf, atile,
             rsem, ssem, asend_sem, arecv_sem):
        d = lax.axis_index("i")
        left = jnp.mod(d - 1, P)
        right = jnp.mod(d + 1, P)

        barrier = pltpu.get_barrier_semaphore()
        for nbr in (left, right):
            pl.semaphore_signal(barrier, inc=1, device_id=(nbr,),
                                device_id_type=pl.DeviceIdType.MESH)
        pl.semaphore_wait(barrier, 2)

        def partial(c):
            xs = x_ref[pl.ds(c * m_per, m_per), :]
            return jnp.dot(xs, w_ref[...], preferred_element_type=jnp.float32)

        sbuf[0] = partial(jnp.mod(d - 1, P)).astype(COMM_DTYPE)

        acc = None
        for s in range(P - 1):
            slot = s % 2
            rdma = pltpu.make_async_remote_copy(
                src_ref=sbuf.at[slot],
                dst_ref=rbuf.at[s],
                send_sem=ssem.at[slot],
                recv_sem=rsem.at[s],
                device_id=(right,),
                device_id_type=pl.DeviceIdType.MESH,
            )
            rdma.start()
            rdma.wait()
            rc = jnp.mod(d - 2 - s, P)
            acc = rbuf[s].astype(jnp.float32) + partial(rc)
            if s < P - 2:
                sbuf[(s + 1) % 2] = acc.astype(COMM_DTYPE)

        y = jnp.maximum(acc, 0.0)
        my_amax = jnp.max(y)

        atile[...] = jnp.full((8, 128), my_amax, jnp.float32)
        abuf[0] = atile[...]
        descs = []
        for o in range(1, P):
            desc = pltpu.make_async_remote_copy(
                src_ref=atile,
                dst_ref=abuf.at[o],
                send_sem=asend_sem.at[o - 1],
                recv_sem=arecv_sem.at[o - 1],
                device_id=(jnp.mod(d + o, P),),
                device_id_type=pl.DeviceIdType.MESH,
            )
            desc.start()
            descs.append(desc)
        for desc in descs:
            desc.wait_send()
        for desc in descs:
            desc.wait_recv()
        amax = jnp.max(abuf[...])

        scale = amax / 448.0
        q = jnp.minimum(y / scale, 448.0).astype(jnp.float8_e4m3fn)
        out_ref[...] = q.astype(jnp.float32) * scale

    return pl.pallas_call(
        body,
        out_shape=jax.ShapeDtypeStruct((m_per, n), jnp.float32),
        in_specs=[
            pl.BlockSpec(memory_space=pltpu.VMEM),
            pl.BlockSpec(memory_space=pltpu.VMEM),
        ],
        out_specs=pl.BlockSpec(memory_space=pltpu.VMEM),
        scratch_shapes=[
            pltpu.VMEM((P - 1, m_per, n), COMM_DTYPE),
            pltpu.VMEM((2, m_per, n), COMM_DTYPE),
            pltpu.VMEM((P, 8, 128), jnp.float32),
            pltpu.VMEM((8, 128), jnp.float32),
            pltpu.SemaphoreType.DMA((P - 1,)),
            pltpu.SemaphoreType.DMA((2,)),
            pltpu.SemaphoreType.DMA((P - 1,)),
            pltpu.SemaphoreType.DMA((P - 1,)),
        ],
        compiler_params=pltpu.CompilerParams(collective_id=0),
    )(x, w_mat)


# baseline (device time: 390219 ns/iter reference)
import jax
import jax.numpy as jnp
from jax import lax
from jax.experimental import pallas as pl
from jax.experimental.pallas import tpu as pltpu

P = 16
COMM_DTYPE = jnp.float32


def kernel(x, w_mat):
    m_full, k_per = x.shape
    _, n = w_mat.shape
    m_per = m_full // P

    def body(x_ref, w_ref, out_ref, rbuf, sbuf, abuf, atile,
             rsem, ssem, asend_sem, arecv_sem):
        d = lax.axis_index("i")
        left = jnp.mod(d - 1, P)
        right = jnp.mod(d + 1, P)

        barrier = pltpu.get_barrier_semaphore()
        for nbr in (left, right):
            pl.semaphore_signal(barrier, inc=1, device_id=(nbr,),
                                device_id_type=pl.DeviceIdType.MESH)
        pl.semaphore_wait(barrier, 2)

        def partial(c):
            xs = x_ref[pl.ds(c * m_per, m_per), :]
            return jnp.dot(xs, w_ref[...], preferred_element_type=jnp.float32)

        sbuf[0] = partial(jnp.mod(d - 1, P)).astype(COMM_DTYPE)

        acc = None
        for s in range(P - 1):
            slot = s % 2
            rdma = pltpu.make_async_remote_copy(
                src_ref=sbuf.at[slot],
                dst_ref=rbuf.at[s],
                send_sem=ssem.at[slot],
                recv_sem=rsem.at[s],
                device_id=(right,),
                device_id_type=pl.DeviceIdType.MESH,
            )
            rdma.start()
            rdma.wait()
            rc = jnp.mod(d - 2 - s, P)
            acc = rbuf[s].astype(jnp.float32) + partial(rc)
            if s < P - 2:
                sbuf[(s + 1) % 2] = acc.astype(COMM_DTYPE)

        y = jnp.maximum(acc, 0.0)
        my_amax = jnp.max(y)

        atile[...] = jnp.full((8, 128), my_amax, jnp.float32)
        abuf[0] = atile[...]
        descs = []
        for o in range(1, P):
            desc = pltpu.make_async_remote_copy(
                src_ref=atile,
                dst_ref=abuf.at[o],
                send_sem=asend_sem.at[o - 1],
                recv_sem=arecv_sem.at[o - 1],
                device_id=(jnp.mod(d + o, P),),
                device_id_type=pl.DeviceIdType.MESH,
            )
            desc.start()
            descs.append(desc)
        for desc in descs:
            desc.wait_send()
        for desc in descs:
            desc.wait_recv()
        amax = jnp.max(abuf[...])

        scale = amax / 448.0
        q = jnp.minimum(y / scale, 448.0).astype(jnp.float8_e4m3fn)
        out_ref[...] = q.astype(jnp.float32) * scale

    return pl.pallas_call(
        body,
        out_shape=jax.ShapeDtypeStruct((m_per, n), jnp.float32),
        in_specs=[
            pl.BlockSpec(memory_space=pltpu.VMEM),
            pl.BlockSpec(memory_space=pltpu.VMEM),
        ],
        out_specs=pl.BlockSpec(memory_space=pltpu.VMEM),
        scratch_shapes=[
            pltpu.VMEM((P - 1, m_per, n), COMM_DTYPE),
            pltpu.VMEM((2, m_per, n), COMM_DTYPE),
            pltpu.VMEM((P, 8, 128), jnp.float32),
            pltpu.VMEM((8, 128), jnp.float32),
            pltpu.SemaphoreType.DMA((P - 1,)),
            pltpu.SemaphoreType.DMA((2,)),
            pltpu.SemaphoreType.DMA((P - 1,)),
            pltpu.SemaphoreType.DMA((P - 1,)),
        ],
        compiler_params=pltpu.CompilerParams(
            collective_id=0, vmem_limit_bytes=100 * 1024 * 1024
        ),
    )(x, w_mat)


# device time: 169990 ns/iter; 2.2955x vs baseline; 2.2955x over previous
import jax
import jax.numpy as jnp
from jax import lax
from jax.experimental import pallas as pl
from jax.experimental.pallas import tpu as pltpu

P = 16


def kernel(x, w_mat):
    m_full, k_per = x.shape
    _, n = w_mat.shape
    m_per = m_full // P
    nh = n // 2

    def body(x_ref, w_ref, out_ref, xbf, wbfA, wbfB, xrow, rbufR, rbufL,
             abuf, atile, rsemR, rsemL, fsendR, fsendL, xsend_sem,
             xrecv_sem, asend_sem, arecv_sem):
        d = lax.axis_index("i")
        left = jnp.mod(d - 1, P)
        right = jnp.mod(d + 1, P)

        xbf[...] = x_ref[...].astype(jnp.bfloat16)
        wbfA[...] = w_ref[:, :nh].astype(jnp.bfloat16)
        wbfB[...] = w_ref[:, nh:].astype(jnp.bfloat16)

        barrier = pltpu.get_barrier_semaphore()
        for o in range(1, P):
            pl.semaphore_signal(barrier, inc=1,
                                device_id=(jnp.mod(d + o, P),),
                                device_id_type=pl.DeviceIdType.MESH)
        pl.semaphore_wait(barrier, P - 1)

        xdescs = []
        for o in range(1, P):
            t = jnp.mod(d + o, P)
            desc = pltpu.make_async_remote_copy(
                src_ref=xbf.at[pl.ds(t * m_per, m_per), :],
                dst_ref=xrow.at[o],
                send_sem=xsend_sem.at[o - 1],
                recv_sem=xrecv_sem.at[o - 1],
                device_id=(t,),
                device_id_type=pl.DeviceIdType.MESH,
            )
            desc.start()
            xdescs.append(desc)

        descsR = [
            pltpu.make_async_remote_copy(
                src_ref=wbfA,
                dst_ref=rbufR.at[0],
                send_sem=fsendR.at[0],
                recv_sem=rsemR.at[0],
                device_id=(right,),
                device_id_type=pl.DeviceIdType.MESH,
            )
        ]
        descsL = [
            pltpu.make_async_remote_copy(
                src_ref=wbfB,
                dst_ref=rbufL.at[0],
                send_sem=fsendL.at[0],
                recv_sem=rsemL.at[0],
                device_id=(left,),
                device_id_type=pl.DeviceIdType.MESH,
            )
        ]
        descsR[0].start()
        descsL[0].start()

        xown = xbf[pl.ds(d * m_per, m_per), :]
        accA = jnp.dot(xown, wbfA[...], preferred_element_type=jnp.float32)
        accB = jnp.dot(xown, wbfB[...], preferred_element_type=jnp.float32)

        x_waited = set()
        for s in range(P - 1):
            descsR[s].wait_recv()
            descsL[s].wait_recv()
            if s < P - 2:
                descR = pltpu.make_async_remote_copy(
                    src_ref=rbufR.at[s],
                    dst_ref=rbufR.at[s + 1],
                    send_sem=fsendR.at[s + 1],
                    recv_sem=rsemR.at[s + 1],
                    device_id=(right,),
                    device_id_type=pl.DeviceIdType.MESH,
                )
                descR.start()
                descsR.append(descR)
                descL = pltpu.make_async_remote_copy(
                    src_ref=rbufL.at[s],
                    dst_ref=rbufL.at[s + 1],
                    send_sem=fsendL.at[s + 1],
                    recv_sem=rsemL.at[s + 1],
                    device_id=(left,),
                    device_id_type=pl.DeviceIdType.MESH,
                )
                descL.start()
                descsL.append(descL)

            for o in (s + 1, P - 1 - s):
                if o not in x_waited:
                    xdescs[o - 1].wait_recv()
                    x_waited.add(o)
            accA = accA + jnp.dot(xrow[s + 1], rbufR[s],
                                  preferred_element_type=jnp.float32)
            accB = accB + jnp.dot(xrow[P - 1 - s], rbufL[s],
                                  preferred_element_type=jnp.float32)

        for desc in xdescs + descsR + descsL:
            desc.wait_send()

        my_amax = jnp.maximum(jnp.maximum(jnp.max(accA), jnp.max(accB)), 0.0)

        atile[...] = jnp.full((8, 128), my_amax, jnp.float32)
        abuf[0] = atile[...]
        adescs = []
        for o in range(1, P):
            desc = pltpu.make_async_remote_copy(
                src_ref=atile,
                dst_ref=abuf.at[o],
                send_sem=asend_sem.at[o - 1],
                recv_sem=arecv_sem.at[o - 1],
                device_id=(jnp.mod(d + o, P),),
                device_id_type=pl.DeviceIdType.MESH,
            )
            desc.start()
            adescs.append(desc)
        for desc in adescs:
            desc.wait_send()
        for desc in adescs:
            desc.wait_recv()
        amax = jnp.max(abuf[...])

        scale = amax / 448.0
        yA = jnp.maximum(accA, 0.0)
        yB = jnp.maximum(accB, 0.0)
        qA = jnp.minimum(yA / scale, 448.0).astype(jnp.float8_e4m3fn)
        qB = jnp.minimum(yB / scale, 448.0).astype(jnp.float8_e4m3fn)
        out_ref[:, :nh] = qA.astype(jnp.float32) * scale
        out_ref[:, nh:] = qB.astype(jnp.float32) * scale

    return pl.pallas_call(
        body,
        out_shape=jax.ShapeDtypeStruct((m_per, n), jnp.float32),
        in_specs=[
            pl.BlockSpec(memory_space=pltpu.VMEM),
            pl.BlockSpec(memory_space=pltpu.VMEM),
        ],
        out_specs=pl.BlockSpec(memory_space=pltpu.VMEM),
        scratch_shapes=[
            pltpu.VMEM((m_full, k_per), jnp.bfloat16),
            pltpu.VMEM((k_per, nh), jnp.bfloat16),
            pltpu.VMEM((k_per, nh), jnp.bfloat16),
            pltpu.VMEM((P, m_per, k_per), jnp.bfloat16),
            pltpu.VMEM((P - 1, k_per, nh), jnp.bfloat16),
            pltpu.VMEM((P - 1, k_per, nh), jnp.bfloat16),
            pltpu.VMEM((P, 8, 128), jnp.float32),
            pltpu.VMEM((8, 128), jnp.float32),
            pltpu.SemaphoreType.DMA((P - 1,)),
            pltpu.SemaphoreType.DMA((P - 1,)),
            pltpu.SemaphoreType.DMA((P - 1,)),
            pltpu.SemaphoreType.DMA((P - 1,)),
            pltpu.SemaphoreType.DMA((P - 1,)),
            pltpu.SemaphoreType.DMA((P - 1,)),
            pltpu.SemaphoreType.DMA((P - 1,)),
            pltpu.SemaphoreType.DMA((P - 1,)),
        ],
        compiler_params=pltpu.CompilerParams(
            collective_id=0, vmem_limit_bytes=100 * 1024 * 1024
        ),
    )(x, w_mat)


# device time: 141462 ns/iter; 2.7585x vs baseline; 1.2017x over previous
import jax
import jax.numpy as jnp
from jax import lax
from jax.experimental import pallas as pl
from jax.experimental.pallas import tpu as pltpu

P = 16
NSTREAM = 4


def kernel(x, w_mat):
    m_full, k_per = x.shape
    _, n = w_mat.shape
    m_per = m_full // P
    nq = n // NSTREAM

    def body(x_ref, w_ref, out_ref, xbf, wbf, xrow,
             rbuf0, rbuf1, rbuf2, rbuf3,
             abuf, atile,
             rsem0, rsem1, rsem2, rsem3,
             fsend0, fsend1, fsend2, fsend3,
             xsend_sem, xrecv_sem, asend_sem, arecv_sem):
        d = lax.axis_index("i")
        left = jnp.mod(d - 1, P)
        right = jnp.mod(d + 1, P)

        rbufs = [rbuf0, rbuf1, rbuf2, rbuf3]
        rsems = [rsem0, rsem1, rsem2, rsem3]
        fsends = [fsend0, fsend1, fsend2, fsend3]
        sdev = [right, right, left, left]

        xbf[...] = x_ref[...].astype(jnp.bfloat16)
        wbf[...] = w_ref[...].astype(jnp.bfloat16)

        barrier = pltpu.get_barrier_semaphore()
        for o in range(1, P):
            pl.semaphore_signal(barrier, inc=1,
                                device_id=(jnp.mod(d + o, P),),
                                device_id_type=pl.DeviceIdType.MESH)
        pl.semaphore_wait(barrier, P - 1)

        xdescs = []
        for o in range(1, P):
            t = jnp.mod(d + o, P)
            desc = pltpu.make_async_remote_copy(
                src_ref=xbf.at[pl.ds(t * m_per, m_per), :],
                dst_ref=xrow.at[o],
                send_sem=xsend_sem.at[o - 1],
                recv_sem=xrecv_sem.at[o - 1],
                device_id=(t,),
                device_id_type=pl.DeviceIdType.MESH,
            )
            desc.start()
            xdescs.append(desc)

        descs = [[], [], [], []]
        for q in range(NSTREAM):
            desc = pltpu.make_async_remote_copy(
                src_ref=wbf.at[:, pl.ds(q * nq, nq)],
                dst_ref=rbufs[q].at[0],
                send_sem=fsends[q].at[0],
                recv_sem=rsems[q].at[0],
                device_id=(sdev[q],),
                device_id_type=pl.DeviceIdType.MESH,
            )
            desc.start()
            descs[q].append(desc)

        xown = xbf[pl.ds(d * m_per, m_per), :]
        accs = [
            jnp.dot(xown, wbf[:, q * nq:(q + 1) * nq],
                    preferred_element_type=jnp.float32)
            for q in range(NSTREAM)
        ]

        x_waited = set()
        for s in range(P - 1):
            for q in (0, 2, 1, 3):
                descs[q][s].wait_recv()
                if s < P - 2:
                    desc = pltpu.make_async_remote_copy(
                        src_ref=rbufs[q].at[s],
                        dst_ref=rbufs[q].at[s + 1],
                        send_sem=fsends[q].at[s + 1],
                        recv_sem=rsems[q].at[s + 1],
                        device_id=(sdev[q],),
                        device_id_type=pl.DeviceIdType.MESH,
                    )
                    desc.start()
                    descs[q].append(desc)

            for o in (s + 1, P - 1 - s):
                if o not in x_waited:
                    xdescs[o - 1].wait_recv()
                    x_waited.add(o)
            for q in range(NSTREAM):
                xr = xrow[s + 1] if q < 2 else xrow[P - 1 - s]
                accs[q] = accs[q] + jnp.dot(
                    xr, rbufs[q][s], preferred_element_type=jnp.float32)

        for group in [xdescs] + descs:
            for desc in group:
                desc.wait_send()

        my_amax = jnp.max(jnp.stack([jnp.max(a) for a in accs]))
        my_amax = jnp.maximum(my_amax, 0.0)

        atile[...] = jnp.full((8, 128), my_amax, jnp.float32)
        abuf[0] = atile[...]
        adescs = []
        for o in range(1, P):
            desc = pltpu.make_async_remote_copy(
                src_ref=atile,
                dst_ref=abuf.at[o],
                send_sem=asend_sem.at[o - 1],
                recv_sem=arecv_sem.at[o - 1],
                device_id=(jnp.mod(d + o, P),),
                device_id_type=pl.DeviceIdType.MESH,
            )
            desc.start()
            adescs.append(desc)
        for desc in adescs:
            desc.wait_send()
        for desc in adescs:
            desc.wait_recv()
        amax = jnp.max(abuf[...])

        scale = amax / 448.0
        for q in range(NSTREAM):
            yq = jnp.maximum(accs[q], 0.0)
            qq = jnp.minimum(yq / scale, 448.0).astype(jnp.float8_e4m3fn)
            out_ref[:, q * nq:(q + 1) * nq] = qq.astype(jnp.float32) * scale

    return pl.pallas_call(
        body,
        out_shape=jax.ShapeDtypeStruct((m_per, n), jnp.float32),
        in_specs=[
            pl.BlockSpec(memory_space=pltpu.VMEM),
            pl.BlockSpec(memory_space=pltpu.VMEM),
        ],
        out_specs=pl.BlockSpec(memory_space=pltpu.VMEM),
        scratch_shapes=[
            pltpu.VMEM((m_full, k_per), jnp.bfloat16),
            pltpu.VMEM((k_per, n), jnp.bfloat16),
            pltpu.VMEM((P, m_per, k_per), jnp.bfloat16),
            pltpu.VMEM((P - 1, k_per, nq), jnp.bfloat16),
            pltpu.VMEM((P - 1, k_per, nq), jnp.bfloat16),
            pltpu.VMEM((P - 1, k_per, nq), jnp.bfloat16),
            pltpu.VMEM((P - 1, k_per, nq), jnp.bfloat16),
            pltpu.VMEM((P, 8, 128), jnp.float32),
            pltpu.VMEM((8, 128), jnp.float32),
            pltpu.SemaphoreType.DMA((P - 1,)),
            pltpu.SemaphoreType.DMA((P - 1,)),
            pltpu.SemaphoreType.DMA((P - 1,)),
            pltpu.SemaphoreType.DMA((P - 1,)),
            pltpu.SemaphoreType.DMA((P - 1,)),
            pltpu.SemaphoreType.DMA((P - 1,)),
            pltpu.SemaphoreType.DMA((P - 1,)),
            pltpu.SemaphoreType.DMA((P - 1,)),
            pltpu.SemaphoreType.DMA((P - 1,)),
            pltpu.SemaphoreType.DMA((P - 1,)),
            pltpu.SemaphoreType.DMA((P - 1,)),
            pltpu.SemaphoreType.DMA((P - 1,)),
        ],
        compiler_params=pltpu.CompilerParams(
            collective_id=0, vmem_limit_bytes=100 * 1024 * 1024
        ),
    )(x, w_mat)
